# baseline (speedup 1.0000x reference)
import functools

import jax
import jax.numpy as jnp
from jax import lax
from jax.experimental import pallas as pl
from jax.experimental.pallas import tpu as pltpu
from jax.experimental.pallas import tpu_sc as plsc

POS_EMB_SIZE = 8193
D_WORD_VEC = 1024
BATCH = 4
MAX_LEN = 8192

_NUM_CORES = 2
_NUM_SUBCORES = 16
_NUM_WORKERS = _NUM_CORES * _NUM_SUBCORES
_ROWS_PER_WORKER = MAX_LEN // _NUM_WORKERS
_CHUNK = 32
_NUM_CHUNKS = _ROWS_PER_WORKER // _CHUNK


_WORKER_WORDS = _ROWS_PER_WORKER * D_WORD_VEC


@functools.partial(
    pl.kernel,
    mesh=plsc.VectorSubcoreMesh(core_axis_name="c", subcore_axis_name="s"),
    out_type=jax.ShapeDtypeStruct((BATCH * MAX_LEN * D_WORD_VEC,), jnp.float32),
    scratch_types=[
        pltpu.SemaphoreType.DMA,
        pltpu.SemaphoreType.DMA,
        pltpu.SemaphoreType.DMA,
        pltpu.SemaphoreType.DMA,
    ],
)
def _sc_broadcast_rows(table_hbm, out_hbm, sem0, sem1, sem2, sem3):
    wid = lax.axis_index("s") * _NUM_CORES + lax.axis_index("c")
    base = wid * _ROWS_PER_WORKER
    sems = (sem0, sem1, sem2, sem3)

    copies = []
    for b in range(BATCH):
        cp = pltpu.make_async_copy(
            table_hbm.at[pl.ds((base + 1) * D_WORD_VEC, _WORKER_WORDS)],
            out_hbm.at[pl.ds((b * MAX_LEN + base) * D_WORD_VEC, _WORKER_WORDS)],
            sems[b],
        )
        cp.start()
        copies.append(cp)
    for cp in copies:
        cp.wait()


def kernel(x, table):
    del x
    flat = _sc_broadcast_rows(table.reshape(-1))
    return flat.reshape(BATCH, MAX_LEN, D_WORD_VEC)


# speedup vs baseline: 17.5898x; 17.5898x over previous
import functools

import jax
import jax.numpy as jnp
from jax import lax
from jax.experimental import pallas as pl
from jax.experimental.pallas import tpu as pltpu
from jax.experimental.pallas import tpu_sc as plsc

POS_EMB_SIZE = 8193
D_WORD_VEC = 1024
BATCH = 4
MAX_LEN = 8192

_NUM_CORES = 2
_NUM_SUBCORES = 16
_NUM_WORKERS = _NUM_CORES * _NUM_SUBCORES
_ROWS_PER_WORKER = MAX_LEN // _NUM_WORKERS
_CHUNK = 32
_NUM_CHUNKS = _ROWS_PER_WORKER // _CHUNK


_CHUNK_WORDS = _CHUNK * D_WORD_VEC
_NBUF = 3


@functools.partial(
    pl.kernel,
    mesh=plsc.VectorSubcoreMesh(core_axis_name="c", subcore_axis_name="s"),
    out_type=jax.ShapeDtypeStruct((BATCH * MAX_LEN * D_WORD_VEC,), jnp.float32),
    scratch_types=(
        [pltpu.VMEM((_CHUNK_WORDS,), jnp.float32) for _ in range(_NBUF)]
        + [pltpu.SemaphoreType.DMA for _ in range(2 * _NBUF)]
    ),
)
def _sc_broadcast_rows(table_hbm, out_hbm, *scratch):
    wid = lax.axis_index("s") * _NUM_CORES + lax.axis_index("c")
    base = wid * _ROWS_PER_WORKER

    bufs = scratch[:_NBUF]
    rsems = scratch[_NBUF : 2 * _NBUF]
    wsems = scratch[2 * _NBUF :]
    pending_writes = [None] * _NBUF

    def read_copy(i):
        row0 = base + i * _CHUNK
        return pltpu.make_async_copy(
            table_hbm.at[pl.ds((row0 + 1) * D_WORD_VEC, _CHUNK_WORDS)],
            bufs[i % _NBUF],
            rsems[i % _NBUF],
        )

    for i in range(min(_NBUF, _NUM_CHUNKS)):
        read_copy(i).start()

    for i in range(_NUM_CHUNKS):
        slot = i % _NBUF
        row0 = base + i * _CHUNK
        read_copy(i).wait()
        writes = []
        for b in range(BATCH):
            cp = pltpu.make_async_copy(
                bufs[slot],
                out_hbm.at[pl.ds((b * MAX_LEN + row0) * D_WORD_VEC, _CHUNK_WORDS)],
                wsems[slot],
            )
            cp.start()
            writes.append(cp)
        pending_writes[slot] = writes
        prev = i - 1
        if prev >= 0 and prev + _NBUF < _NUM_CHUNKS:
            pslot = prev % _NBUF
            for cp in pending_writes[pslot]:
                cp.wait()
            pending_writes[pslot] = None
            read_copy(prev + _NBUF).start()

    for slot in range(_NBUF):
        if pending_writes[slot] is not None:
            for cp in pending_writes[slot]:
                cp.wait()


def kernel(x, table):
    del x
    flat = _sc_broadcast_rows(table.reshape(-1))
    return flat.reshape(BATCH, MAX_LEN, D_WORD_VEC)


# speedup vs baseline: 56.5554x; 3.2152x over previous
import functools

import jax
import jax.numpy as jnp
from jax import lax
from jax.experimental import pallas as pl
from jax.experimental.pallas import tpu as pltpu
from jax.experimental.pallas import tpu_sc as plsc

POS_EMB_SIZE = 8193
D_WORD_VEC = 1024
BATCH = 4
MAX_LEN = 8192

_NUM_CORES = 2
_NUM_SUBCORES = 16
_NUM_WORKERS = _NUM_CORES * _NUM_SUBCORES
_ROWS_PER_WORKER = MAX_LEN // _NUM_WORKERS
_CHUNK = 32
_NUM_CHUNKS = _ROWS_PER_WORKER // _CHUNK
_NBUF = 3
_LANES = 16


@functools.partial(
    pl.kernel,
    mesh=plsc.VectorSubcoreMesh(core_axis_name="c", subcore_axis_name="s"),
    out_type=jax.ShapeDtypeStruct((BATCH, MAX_LEN, D_WORD_VEC), jnp.float32),
    scratch_types=(
        [pltpu.VMEM((_CHUNK, D_WORD_VEC), jnp.float32) for _ in range(_NBUF)]
        + [pltpu.VMEM((_ROWS_PER_WORKER,), jnp.int32)]
        + [pltpu.SemaphoreType.DMA for _ in range(2 * _NBUF)]
    ),
)
def _sc_broadcast_rows(table_hbm, out_hbm, *scratch):
    wid = lax.axis_index("s") * _NUM_CORES + lax.axis_index("c")
    base = wid * _ROWS_PER_WORKER

    bufs = scratch[:_NBUF]
    idx = scratch[_NBUF]
    rsems = scratch[_NBUF + 1 : 2 * _NBUF + 1]
    wsems = scratch[2 * _NBUF + 1 :]
    pending_writes = [None] * _NBUF

    lane = lax.iota(jnp.int32, _LANES)
    for j in range(_ROWS_PER_WORKER // _LANES):
        idx[pl.ds(j * _LANES, _LANES)] = lane + (base + 1 + j * _LANES)

    def gather_copy(i):
        return pltpu.make_async_copy(
            table_hbm.at[idx.at[pl.ds(i * _CHUNK, _CHUNK)]],
            bufs[i % _NBUF],
            rsems[i % _NBUF],
        )

    for i in range(min(_NBUF, _NUM_CHUNKS)):
        gather_copy(i).start()

    for i in range(_NUM_CHUNKS):
        slot = i % _NBUF
        row0 = base + i * _CHUNK
        gather_copy(i).wait()
        writes = []
        for b in range(BATCH):
            cp = pltpu.make_async_copy(
                bufs[slot],
                out_hbm.at[b, pl.ds(row0, _CHUNK)],
                wsems[slot],
            )
            cp.start()
            writes.append(cp)
        pending_writes[slot] = writes
        prev = i - 1
        if prev >= 0 and prev + _NBUF < _NUM_CHUNKS:
            pslot = prev % _NBUF
            for cp in pending_writes[pslot]:
                cp.wait()
            pending_writes[pslot] = None
            gather_copy(prev + _NBUF).start()

    for slot in range(_NBUF):
        if pending_writes[slot] is not None:
            for cp in pending_writes[slot]:
                cp.wait()


def kernel(x, table):
    del x
    return _sc_broadcast_rows(table)
